# baseline (speedup 1.0000x reference)
import functools

import jax
import jax.numpy as jnp
from jax import lax
from jax.experimental import pallas as pl
from jax.experimental.pallas import tpu as pltpu
from jax.experimental.pallas import tpu_sc as plsc

_NA = 3
_NC = 80
_ANCHOR_T = 4.0
_BOX_GAIN = 0.05
_CLS_GAIN = 0.5
_AG = (
    ((10.0 / 8.0, 13.0 / 8.0), (16.0 / 8.0, 30.0 / 8.0), (33.0 / 8.0, 23.0 / 8.0)),
    ((30.0 / 16.0, 61.0 / 16.0), (62.0 / 16.0, 45.0 / 16.0), (59.0 / 16.0, 119.0 / 16.0)),
    ((116.0 / 32.0, 90.0 / 32.0), (156.0 / 32.0, 198.0 / 32.0), (373.0 / 32.0, 326.0 / 32.0)),
)
_GRIDS = ((80, 80), (40, 40), (20, 20))
_B = 16
_M = 300
_MP = 320
_NPAD = 1280
_RPW = _NPAD // 32


def _meta_body(t_ref, i0_ref, i1_ref, i2_ref, s0_ref, s1_ref, s2_ref):
    tr = t_ref[...]
    col = lax.broadcasted_iota(jnp.int32, (8, _MP), 1)
    arow = lax.broadcasted_iota(jnp.int32, (8, _MP), 0)
    ok = (col < _M) & (arow < _NA)
    b = tr[0:1, :]
    x = tr[2:3, :]
    y = tr[3:4, :]
    bi = b.astype(jnp.int32)
    for (ny, nx), o_ref, s_ref in zip(
            _GRIDS, (i0_ref, i1_ref, i2_ref), (s0_ref, s1_ref, s2_ref)):
        gi = jnp.clip((x * nx).astype(jnp.int32), 0, nx - 1)
        gj = jnp.clip((y * ny).astype(jnp.int32), 0, ny - 1)
        idx = ((bi * _NA + arow) * ny + gj) * nx + gi
        o_ref[...] = jnp.where(ok, lax.shift_right_logical(idx, 3), 0)
        s_ref[...] = jnp.where(ok, jnp.bitwise_and(idx, 7), 0)


def _build_indices(t_pad):
    return pl.pallas_call(
        _meta_body,
        out_shape=[jax.ShapeDtypeStruct((8, _MP), jnp.int32)] * 6,
    )(t_pad)


_GQ = 16
_LSTEPS = _NPAD // _GQ


def _gather_all(tbls, tidx_all, sub_all):
    del sub_all

    def body(tref, *refs):
        o_ref = refs[3 * _GQ]
        i = pl.program_id(0)
        for lvl in range(3):
            @pl.when(i // _LSTEPS == lvl)
            def _(lvl=lvl):
                for q in range(_GQ):
                    o_ref[q] = refs[lvl * _GQ + q][0]

    def imap(lvl, q):
        def f(i, tref):
            j = jnp.clip(i - lvl * _LSTEPS, 0, _LSTEPS - 1)
            return (tref[(lvl * _LSTEPS + j) * _GQ + q], 0, 0)
        return f

    in_specs = [pl.BlockSpec((1, 8, 85), imap(lvl, q))
                for lvl in range(3) for q in range(_GQ)]
    ins = [t for t in tbls for _ in range(_GQ)]
    return pl.pallas_call(
        body,
        grid_spec=pltpu.PrefetchScalarGridSpec(
            num_scalar_prefetch=1,
            grid=(3 * _LSTEPS,),
            in_specs=in_specs,
            out_specs=pl.BlockSpec((_GQ, 8, 85), lambda i, tref: (i, 0, 0)),
        ),
        out_shape=jax.ShapeDtypeStruct((3 * _NPAD, 8, 85), jnp.float32),
    )(tidx_all, *ins)


def _obj_softplus_sum(tbl, blk):
    n = tbl.shape[0]

    def body(x_ref, o_ref):
        x = x_ref[:, 4:5]
        s = jnp.sum(jnp.maximum(x, 0.0) + jnp.log1p(jnp.exp(-jnp.abs(x))))

        @pl.when(pl.program_id(0) == 0)
        def _():
            o_ref[0, 0] = 0.0

        o_ref[0, 0] += s

    return pl.pallas_call(
        body,
        grid=(n // blk,),
        in_specs=[pl.BlockSpec((blk, 85), lambda i: (i, 0))],
        out_specs=pl.BlockSpec(memory_space=pltpu.SMEM),
        out_shape=jax.ShapeDtypeStruct((1, 1), jnp.float32),
    )(tbl)


def _sparse_body(t_ref, g_ref, s0_ref, s1_ref, s2_ref,
                 loss_ref, lbox_ref, lobj_ref, lcls_ref):
    tr = t_ref[...]
    colv = lax.broadcasted_iota(jnp.int32, (1, _MP), 1)
    valid = colv < _M
    b = tr[0:1, :]
    cls = tr[1:2, :]
    x = tr[2:3, :]
    y = tr[3:4, :]
    w = tr[4:5, :]
    h = tr[5:6, :]
    bi = b.astype(jnp.int32)
    tcls = cls.astype(jnp.int32)
    arow3 = lax.broadcasted_iota(jnp.int32, (_NA, _MP), 0)
    order = arow3 * _M + lax.broadcasted_iota(jnp.int32, (_NA, _MP), 1)

    lbox = jnp.float32(0.0)
    lobj = jnp.float32(0.0)
    lcls = jnp.float32(0.0)
    for lvl, (ny, nx) in enumerate(_GRIDS):
        gx = x * nx
        gy = y * ny
        gw = w * nx
        gh = h * ny
        ar = lax.broadcasted_iota(jnp.int32, (_NA, 1), 0)
        ax0, ax1, ax2 = (_AG[lvl][a][0] for a in range(_NA))
        ay0, ay1, ay2 = (_AG[lvl][a][1] for a in range(_NA))
        awx = jnp.where(ar == 0, ax0, jnp.where(ar == 1, ax1, ax2))
        awy = jnp.where(ar == 0, ay0, jnp.where(ar == 1, ay1, ay2))
        rx = gw / awx
        ry = gh / awy
        mr = jnp.maximum(jnp.maximum(rx, 1.0 / (rx + 1e-9)),
                         jnp.maximum(ry, 1.0 / (ry + 1e-9)))
        mask = (mr < _ANCHOR_T) & valid
        mf = mask.astype(jnp.float32)
        cnt = jnp.sum(mf)
        gi = jnp.clip((x * nx).astype(jnp.int32), 0, nx - 1)
        gj = jnp.clip((y * ny).astype(jnp.int32), 0, ny - 1)
        gif = gi.astype(jnp.float32)
        gjf = gj.astype(jnp.float32)

        cell = ((bi * _NA + arow3) * ny + gj) * nx + gi
        sub = jnp.bitwise_and(cell, 7)
        ps = jnp.zeros((_NA, _MP, 85), jnp.float32)
        for s in range(8):
            sel = (sub == s).astype(jnp.float32)
            ps = ps + g_ref[lvl, 0:_NA, :, s, :] * sel[:, :, None]

        sig = jax.nn.sigmoid
        pxy_x = sig(ps[:, :, 0]) * 2.0 - 0.5 + gif
        pxy_y = sig(ps[:, :, 1]) * 2.0 - 0.5 + gjf
        pwh_x = (sig(ps[:, :, 2]) * 2.0) ** 2 * awx
        pwh_y = (sig(ps[:, :, 3]) * 2.0) ** 2 * awy

        b1x1 = pxy_x - pwh_x * 0.5
        b1y1 = pxy_y - pwh_y * 0.5
        b1x2 = pxy_x + pwh_x * 0.5
        b1y2 = pxy_y + pwh_y * 0.5
        b2x1 = gx - gw * 0.5
        b2y1 = gy - gh * 0.5
        b2x2 = gx + gw * 0.5
        b2y2 = gy + gh * 0.5
        iw = jnp.clip(jnp.minimum(b1x2, b2x2) - jnp.maximum(b1x1, b2x1), 0.0)
        ih = jnp.clip(jnp.minimum(b1y2, b2y2) - jnp.maximum(b1y1, b2y1), 0.0)
        inter = iw * ih
        a1 = jnp.clip(b1x2 - b1x1, 0.0) * jnp.clip(b1y2 - b1y1, 0.0)
        a2 = jnp.clip(b2x2 - b2x1, 0.0) * jnp.clip(b2y2 - b2y1, 0.0)
        iou = inter / (a1 + a2 - inter + 1e-7)
        lbox = lbox + jnp.where(cnt > 0, jnp.sum((1.0 - iou) * mf) / cnt, 0.0)

        logits = ps[:, :, 5:]
        chan = lax.broadcasted_iota(jnp.int32, (_NA, _MP, _NC), 2)
        tmat = (chan == tcls[0][None, :, None]).astype(jnp.float32)
        per = (jnp.maximum(logits, 0.0) - logits * tmat
               + jnp.log1p(jnp.exp(-jnp.abs(logits))))
        lcls = lcls + jnp.where(
            cnt > 0, jnp.sum(per * mf[:, :, None]) / (cnt * _NC), 0.0)

        overwritten = jnp.zeros((_NA, _MP), jnp.bool_)
        for aj in range(_NA):
            cj = cell[aj:aj + 1, :]
            ej = mask[aj:aj + 1, :]
            oj = order[aj:aj + 1, :]
            eq = cell[:, :, None] == cj[0][None, None, :]
            later = oj[0][None, None, :] > order[:, :, None]
            hit = eq & later & ej[0][None, None, :]
            overwritten = overwritten | jnp.any(hit, axis=2)
        win = mask & ~overwritten
        corr = jnp.sum(jnp.where(win, ps[:, :, 4] * jnp.clip(iou, 0.0), 0.0))
        ncell = _B * _NA * ny * nx
        sref = (s0_ref, s1_ref, s2_ref)[lvl]
        lobj = lobj + (sref[0, 0] - corr) / ncell

    loss_ref[0, 0] = _BOX_GAIN * lbox + lobj + _CLS_GAIN * lcls
    lbox_ref[0, 0] = lbox
    lobj_ref[0, 0] = lobj
    lcls_ref[0, 0] = lcls


def _sparse_call(t_pad, g, s0, s1, s2):
    smem = pl.BlockSpec(memory_space=pltpu.SMEM)
    return pl.pallas_call(
        _sparse_body,
        in_specs=[pl.BlockSpec()] * 2 + [smem] * 3,
        out_specs=[smem] * 4,
        out_shape=[jax.ShapeDtypeStruct((1, 1), jnp.float32)] * 4,
    )(t_pad, g, s0, s1, s2)


def kernel(p0, p1, p2, targets):
    t_pad = jnp.zeros((8, _MP), jnp.float32).at[:6, :_M].set(targets.T)
    idx0, idx1, idx2, sub0, sub1, sub2 = _build_indices(t_pad)
    flat = lambda ix: ix[:4].reshape(_NPAD)
    tidx_all = jnp.concatenate([flat(idx0), flat(idx1), flat(idx2)])
    sub_all = jnp.concatenate([flat(sub0), flat(sub1), flat(sub2)])
    tbls = [p.reshape(-1, 8, 85) for p in (p0, p1, p2)]
    g = _gather_all(tbls, tidx_all, sub_all)
    flats = [p.reshape(-1, 85) for p in (p0, p1, p2)]
    s0 = _obj_softplus_sum(flats[0], 6400)
    s1 = _obj_softplus_sum(flats[1], 6400)
    s2 = _obj_softplus_sum(flats[2], 6400)
    loss, lbox, lobj, lcls = _sparse_call(
        t_pad, g.reshape(3, 4, _MP, 8, 85), s0, s1, s2)
    return (loss[0, 0], lbox[0, 0], lobj[0, 0], lcls[0, 0])


# speedup vs baseline: 1.2821x; 1.2821x over previous
import functools

import jax
import jax.numpy as jnp
from jax import lax
from jax.experimental import pallas as pl
from jax.experimental.pallas import tpu as pltpu
from jax.experimental.pallas import tpu_sc as plsc

_NA = 3
_NC = 80
_ANCHOR_T = 4.0
_BOX_GAIN = 0.05
_CLS_GAIN = 0.5
_AG = (
    ((10.0 / 8.0, 13.0 / 8.0), (16.0 / 8.0, 30.0 / 8.0), (33.0 / 8.0, 23.0 / 8.0)),
    ((30.0 / 16.0, 61.0 / 16.0), (62.0 / 16.0, 45.0 / 16.0), (59.0 / 16.0, 119.0 / 16.0)),
    ((116.0 / 32.0, 90.0 / 32.0), (156.0 / 32.0, 198.0 / 32.0), (373.0 / 32.0, 326.0 / 32.0)),
)
_GRIDS = ((80, 80), (40, 40), (20, 20))
_B = 16
_M = 300
_MP = 320
_NPAD = 1280
_RPW = _NPAD // 32


def _meta_body(t_ref, i0_ref, i1_ref, i2_ref, s0_ref, s1_ref, s2_ref):
    tr = t_ref[...]
    col = lax.broadcasted_iota(jnp.int32, (8, _MP), 1)
    arow = lax.broadcasted_iota(jnp.int32, (8, _MP), 0)
    ok = (col < _M) & (arow < _NA)
    b = tr[0:1, :]
    x = tr[2:3, :]
    y = tr[3:4, :]
    bi = b.astype(jnp.int32)
    for (ny, nx), o_ref, s_ref in zip(
            _GRIDS, (i0_ref, i1_ref, i2_ref), (s0_ref, s1_ref, s2_ref)):
        gi = jnp.clip((x * nx).astype(jnp.int32), 0, nx - 1)
        gj = jnp.clip((y * ny).astype(jnp.int32), 0, ny - 1)
        idx = ((bi * _NA + arow) * ny + gj) * nx + gi
        o_ref[...] = jnp.where(ok, lax.shift_right_logical(idx, 3), 0)
        s_ref[...] = jnp.where(ok, jnp.bitwise_and(idx, 7), 0)


def _build_indices(t_pad):
    return pl.pallas_call(
        _meta_body,
        out_shape=[jax.ShapeDtypeStruct((8, _MP), jnp.int32)] * 6,
    )(t_pad)


_GQ = 16
_LSTEPS = _NPAD // _GQ


def _gather_all(tbls, tidx_all, sub_all):
    del sub_all

    def body(tref, *refs):
        o_ref = refs[_GQ]
        for q in range(_GQ):
            o_ref[q] = refs[q][0]

    def imap(q):
        return lambda i, tref: (tref[i * _GQ + q], 0, 0)

    def one(tbl, tidx):
        return pl.pallas_call(
            body,
            grid_spec=pltpu.PrefetchScalarGridSpec(
                num_scalar_prefetch=1,
                grid=(_NPAD // _GQ,),
                in_specs=[pl.BlockSpec((1, 8, 85), imap(q))
                          for q in range(_GQ)],
                out_specs=pl.BlockSpec((_GQ, 8, 85), lambda i, tref: (i, 0, 0)),
            ),
            out_shape=jax.ShapeDtypeStruct((_NPAD, 8, 85), jnp.float32),
        )(tidx, *([tbl] * _GQ))

    return [one(t, tidx_all[lvl * _NPAD:(lvl + 1) * _NPAD])
            for lvl, t in enumerate(tbls)]


def _obj_softplus_sum(tbl, blk):
    n = tbl.shape[0]

    def body(x_ref, o_ref):
        x = x_ref[:, 4:5]
        s = jnp.sum(jnp.maximum(x, 0.0) + jnp.log1p(jnp.exp(-jnp.abs(x))))

        @pl.when(pl.program_id(0) == 0)
        def _():
            o_ref[0, 0] = 0.0

        o_ref[0, 0] += s

    return pl.pallas_call(
        body,
        grid=(n // blk,),
        in_specs=[pl.BlockSpec((blk, 85), lambda i: (i, 0))],
        out_specs=pl.BlockSpec(memory_space=pltpu.SMEM),
        out_shape=jax.ShapeDtypeStruct((1, 1), jnp.float32),
    )(tbl)


def _sparse_body(t_ref, g0_ref, g1_ref, g2_ref, s0_ref, s1_ref, s2_ref,
                 loss_ref, lbox_ref, lobj_ref, lcls_ref):
    tr = t_ref[...]
    colv = lax.broadcasted_iota(jnp.int32, (1, _MP), 1)
    valid = colv < _M
    b = tr[0:1, :]
    cls = tr[1:2, :]
    x = tr[2:3, :]
    y = tr[3:4, :]
    w = tr[4:5, :]
    h = tr[5:6, :]
    bi = b.astype(jnp.int32)
    tcls = cls.astype(jnp.int32)
    arow3 = lax.broadcasted_iota(jnp.int32, (_NA, _MP), 0)
    order = arow3 * _M + lax.broadcasted_iota(jnp.int32, (_NA, _MP), 1)

    lbox = jnp.float32(0.0)
    lobj = jnp.float32(0.0)
    lcls = jnp.float32(0.0)
    for lvl, (ny, nx) in enumerate(_GRIDS):
        gx = x * nx
        gy = y * ny
        gw = w * nx
        gh = h * ny
        ar = lax.broadcasted_iota(jnp.int32, (_NA, 1), 0)
        ax0, ax1, ax2 = (_AG[lvl][a][0] for a in range(_NA))
        ay0, ay1, ay2 = (_AG[lvl][a][1] for a in range(_NA))
        awx = jnp.where(ar == 0, ax0, jnp.where(ar == 1, ax1, ax2))
        awy = jnp.where(ar == 0, ay0, jnp.where(ar == 1, ay1, ay2))
        rx = gw / awx
        ry = gh / awy
        mr = jnp.maximum(jnp.maximum(rx, 1.0 / (rx + 1e-9)),
                         jnp.maximum(ry, 1.0 / (ry + 1e-9)))
        mask = (mr < _ANCHOR_T) & valid
        mf = mask.astype(jnp.float32)
        cnt = jnp.sum(mf)
        gi = jnp.clip((x * nx).astype(jnp.int32), 0, nx - 1)
        gj = jnp.clip((y * ny).astype(jnp.int32), 0, ny - 1)
        gif = gi.astype(jnp.float32)
        gjf = gj.astype(jnp.float32)

        cell = ((bi * _NA + arow3) * ny + gj) * nx + gi
        sub = jnp.bitwise_and(cell, 7)
        g_ref = (g0_ref, g1_ref, g2_ref)[lvl]
        ps = jnp.zeros((_NA, _MP, 85), jnp.float32)
        for s in range(8):
            sel = (sub == s).astype(jnp.float32)
            ps = ps + g_ref[0:_NA, :, s, :] * sel[:, :, None]

        sig = jax.nn.sigmoid
        pxy_x = sig(ps[:, :, 0]) * 2.0 - 0.5 + gif
        pxy_y = sig(ps[:, :, 1]) * 2.0 - 0.5 + gjf
        pwh_x = (sig(ps[:, :, 2]) * 2.0) ** 2 * awx
        pwh_y = (sig(ps[:, :, 3]) * 2.0) ** 2 * awy

        b1x1 = pxy_x - pwh_x * 0.5
        b1y1 = pxy_y - pwh_y * 0.5
        b1x2 = pxy_x + pwh_x * 0.5
        b1y2 = pxy_y + pwh_y * 0.5
        b2x1 = gx - gw * 0.5
        b2y1 = gy - gh * 0.5
        b2x2 = gx + gw * 0.5
        b2y2 = gy + gh * 0.5
        iw = jnp.clip(jnp.minimum(b1x2, b2x2) - jnp.maximum(b1x1, b2x1), 0.0)
        ih = jnp.clip(jnp.minimum(b1y2, b2y2) - jnp.maximum(b1y1, b2y1), 0.0)
        inter = iw * ih
        a1 = jnp.clip(b1x2 - b1x1, 0.0) * jnp.clip(b1y2 - b1y1, 0.0)
        a2 = jnp.clip(b2x2 - b2x1, 0.0) * jnp.clip(b2y2 - b2y1, 0.0)
        iou = inter / (a1 + a2 - inter + 1e-7)
        lbox = lbox + jnp.where(cnt > 0, jnp.sum((1.0 - iou) * mf) / cnt, 0.0)

        logits = ps[:, :, 5:]
        chan = lax.broadcasted_iota(jnp.int32, (_NA, _MP, _NC), 2)
        tmat = (chan == tcls[0][None, :, None]).astype(jnp.float32)
        per = (jnp.maximum(logits, 0.0) - logits * tmat
               + jnp.log1p(jnp.exp(-jnp.abs(logits))))
        lcls = lcls + jnp.where(
            cnt > 0, jnp.sum(per * mf[:, :, None]) / (cnt * _NC), 0.0)

        overwritten = jnp.zeros((_NA, _MP), jnp.bool_)
        for aj in range(_NA):
            cj = cell[aj:aj + 1, :]
            ej = mask[aj:aj + 1, :]
            oj = order[aj:aj + 1, :]
            eq = cell[:, :, None] == cj[0][None, None, :]
            later = oj[0][None, None, :] > order[:, :, None]
            hit = eq & later & ej[0][None, None, :]
            overwritten = overwritten | jnp.any(hit, axis=2)
        win = mask & ~overwritten
        corr = jnp.sum(jnp.where(win, ps[:, :, 4] * jnp.clip(iou, 0.0), 0.0))
        ncell = _B * _NA * ny * nx
        sref = (s0_ref, s1_ref, s2_ref)[lvl]
        lobj = lobj + (sref[0, 0] - corr) / ncell

    loss_ref[0, 0] = _BOX_GAIN * lbox + lobj + _CLS_GAIN * lcls
    lbox_ref[0, 0] = lbox
    lobj_ref[0, 0] = lobj
    lcls_ref[0, 0] = lcls


def _sparse_call(t_pad, g0, g1, g2, s0, s1, s2):
    smem = pl.BlockSpec(memory_space=pltpu.SMEM)
    return pl.pallas_call(
        _sparse_body,
        in_specs=[pl.BlockSpec()] * 4 + [smem] * 3,
        out_specs=[smem] * 4,
        out_shape=[jax.ShapeDtypeStruct((1, 1), jnp.float32)] * 4,
    )(t_pad, g0, g1, g2, s0, s1, s2)


def kernel(p0, p1, p2, targets):
    t_pad = jnp.zeros((8, _MP), jnp.float32).at[:6, :_M].set(targets.T)
    idx0, idx1, idx2, sub0, sub1, sub2 = _build_indices(t_pad)
    flat = lambda ix: ix[:4].reshape(_NPAD)
    tidx_all = jnp.concatenate([flat(idx0), flat(idx1), flat(idx2)])
    sub_all = jnp.concatenate([flat(sub0), flat(sub1), flat(sub2)])
    tbls = [p.reshape(-1, 8, 85) for p in (p0, p1, p2)]
    g0, g1, g2 = _gather_all(tbls, tidx_all, sub_all)
    flats = [p.reshape(-1, 85) for p in (p0, p1, p2)]
    s0 = _obj_softplus_sum(flats[0], 6400)
    s1 = _obj_softplus_sum(flats[1], 6400)
    s2 = _obj_softplus_sum(flats[2], 6400)
    r = lambda g: g.reshape(4, _MP, 8, 85)
    loss, lbox, lobj, lcls = _sparse_call(
        t_pad, r(g0), r(g1), r(g2), s0, s1, s2)
    return (loss[0, 0], lbox[0, 0], lobj[0, 0], lcls[0, 0])


# speedup vs baseline: 1.4169x; 1.1051x over previous
import functools

import jax
import jax.numpy as jnp
from jax import lax
from jax.experimental import pallas as pl
from jax.experimental.pallas import tpu as pltpu
from jax.experimental.pallas import tpu_sc as plsc

_NA = 3
_NC = 80
_ANCHOR_T = 4.0
_BOX_GAIN = 0.05
_CLS_GAIN = 0.5
_AG = (
    ((10.0 / 8.0, 13.0 / 8.0), (16.0 / 8.0, 30.0 / 8.0), (33.0 / 8.0, 23.0 / 8.0)),
    ((30.0 / 16.0, 61.0 / 16.0), (62.0 / 16.0, 45.0 / 16.0), (59.0 / 16.0, 119.0 / 16.0)),
    ((116.0 / 32.0, 90.0 / 32.0), (156.0 / 32.0, 198.0 / 32.0), (373.0 / 32.0, 326.0 / 32.0)),
)
_GRIDS = ((80, 80), (40, 40), (20, 20))
_B = 16
_M = 300
_MP = 320
_NPAD = 1280
_RPW = _NPAD // 32


def _meta_body(t_ref, i0_ref, i1_ref, i2_ref, s0_ref, s1_ref, s2_ref):
    tr = t_ref[...]
    col = lax.broadcasted_iota(jnp.int32, (8, _MP), 1)
    arow = lax.broadcasted_iota(jnp.int32, (8, _MP), 0)
    ok = (col < _M) & (arow < _NA)
    b = tr[0:1, :]
    x = tr[2:3, :]
    y = tr[3:4, :]
    bi = b.astype(jnp.int32)
    for (ny, nx), o_ref, s_ref in zip(
            _GRIDS, (i0_ref, i1_ref, i2_ref), (s0_ref, s1_ref, s2_ref)):
        gi = jnp.clip((x * nx).astype(jnp.int32), 0, nx - 1)
        gj = jnp.clip((y * ny).astype(jnp.int32), 0, ny - 1)
        idx = ((bi * _NA + arow) * ny + gj) * nx + gi
        o_ref[...] = jnp.where(ok, lax.shift_right_logical(idx, 3), 0)
        s_ref[...] = jnp.where(ok, jnp.bitwise_and(idx, 7), 0)


def _build_indices(t_pad):
    return pl.pallas_call(
        _meta_body,
        out_shape=[jax.ShapeDtypeStruct((8, _MP), jnp.int32)] * 6,
    )(t_pad)


_GQ = 32
_LSTEPS = _NPAD // _GQ


def _gather_all(tbls, tidx_all, sub_all):
    del sub_all

    def body(tref, *refs):
        o_ref = refs[_GQ]
        for q in range(_GQ):
            o_ref[q] = refs[q][0]

    def imap(q):
        return lambda i, tref: (tref[i * _GQ + q], 0, 0)

    def one(tbl, tidx):
        return pl.pallas_call(
            body,
            grid_spec=pltpu.PrefetchScalarGridSpec(
                num_scalar_prefetch=1,
                grid=(_NPAD // _GQ,),
                in_specs=[pl.BlockSpec((1, 8, 85), imap(q))
                          for q in range(_GQ)],
                out_specs=pl.BlockSpec((_GQ, 8, 85), lambda i, tref: (i, 0, 0)),
            ),
            out_shape=jax.ShapeDtypeStruct((_NPAD, 8, 85), jnp.float32),
        )(tidx, *([tbl] * _GQ))

    return [one(t, tidx_all[lvl * _NPAD:(lvl + 1) * _NPAD])
            for lvl, t in enumerate(tbls)]


def _obj_softplus_sum(tbl, blk):
    n = tbl.shape[0]

    def body(x_ref, o_ref):
        x = x_ref[:, 4:5]
        s = jnp.sum(jnp.maximum(x, 0.0) + jnp.log1p(jnp.exp(-jnp.abs(x))))

        @pl.when(pl.program_id(0) == 0)
        def _():
            o_ref[0, 0] = 0.0

        o_ref[0, 0] += s

    return pl.pallas_call(
        body,
        grid=(n // blk,),
        in_specs=[pl.BlockSpec((blk, 85), lambda i: (i, 0))],
        out_specs=pl.BlockSpec(memory_space=pltpu.SMEM),
        out_shape=jax.ShapeDtypeStruct((1, 1), jnp.float32),
    )(tbl)


def _sparse_body(t_ref, g0_ref, g1_ref, g2_ref, s0_ref, s1_ref, s2_ref,
                 loss_ref, lbox_ref, lobj_ref, lcls_ref):
    tr = t_ref[...]
    colv = lax.broadcasted_iota(jnp.int32, (1, _MP), 1)
    valid = colv < _M
    b = tr[0:1, :]
    cls = tr[1:2, :]
    x = tr[2:3, :]
    y = tr[3:4, :]
    w = tr[4:5, :]
    h = tr[5:6, :]
    bi = b.astype(jnp.int32)
    tcls = cls.astype(jnp.int32)
    arow3 = lax.broadcasted_iota(jnp.int32, (_NA, _MP), 0)
    order = arow3 * _M + lax.broadcasted_iota(jnp.int32, (_NA, _MP), 1)

    lbox = jnp.float32(0.0)
    lobj = jnp.float32(0.0)
    lcls = jnp.float32(0.0)
    for lvl, (ny, nx) in enumerate(_GRIDS):
        gx = x * nx
        gy = y * ny
        gw = w * nx
        gh = h * ny
        ar = lax.broadcasted_iota(jnp.int32, (_NA, 1), 0)
        ax0, ax1, ax2 = (_AG[lvl][a][0] for a in range(_NA))
        ay0, ay1, ay2 = (_AG[lvl][a][1] for a in range(_NA))
        awx = jnp.where(ar == 0, ax0, jnp.where(ar == 1, ax1, ax2))
        awy = jnp.where(ar == 0, ay0, jnp.where(ar == 1, ay1, ay2))
        rx = gw / awx
        ry = gh / awy
        mr = jnp.maximum(jnp.maximum(rx, 1.0 / (rx + 1e-9)),
                         jnp.maximum(ry, 1.0 / (ry + 1e-9)))
        mask = (mr < _ANCHOR_T) & valid
        mf = mask.astype(jnp.float32)
        cnt = jnp.sum(mf)
        gi = jnp.clip((x * nx).astype(jnp.int32), 0, nx - 1)
        gj = jnp.clip((y * ny).astype(jnp.int32), 0, ny - 1)
        gif = gi.astype(jnp.float32)
        gjf = gj.astype(jnp.float32)

        cell = ((bi * _NA + arow3) * ny + gj) * nx + gi
        sub = jnp.bitwise_and(cell, 7)
        g_ref = (g0_ref, g1_ref, g2_ref)[lvl]
        ps = jnp.zeros((_NA, _MP, 85), jnp.float32)
        for s in range(8):
            sel = (sub == s).astype(jnp.float32)
            ps = ps + g_ref[0:_NA, :, s, :] * sel[:, :, None]

        sig = jax.nn.sigmoid
        pxy_x = sig(ps[:, :, 0]) * 2.0 - 0.5 + gif
        pxy_y = sig(ps[:, :, 1]) * 2.0 - 0.5 + gjf
        pwh_x = (sig(ps[:, :, 2]) * 2.0) ** 2 * awx
        pwh_y = (sig(ps[:, :, 3]) * 2.0) ** 2 * awy

        b1x1 = pxy_x - pwh_x * 0.5
        b1y1 = pxy_y - pwh_y * 0.5
        b1x2 = pxy_x + pwh_x * 0.5
        b1y2 = pxy_y + pwh_y * 0.5
        b2x1 = gx - gw * 0.5
        b2y1 = gy - gh * 0.5
        b2x2 = gx + gw * 0.5
        b2y2 = gy + gh * 0.5
        iw = jnp.clip(jnp.minimum(b1x2, b2x2) - jnp.maximum(b1x1, b2x1), 0.0)
        ih = jnp.clip(jnp.minimum(b1y2, b2y2) - jnp.maximum(b1y1, b2y1), 0.0)
        inter = iw * ih
        a1 = jnp.clip(b1x2 - b1x1, 0.0) * jnp.clip(b1y2 - b1y1, 0.0)
        a2 = jnp.clip(b2x2 - b2x1, 0.0) * jnp.clip(b2y2 - b2y1, 0.0)
        iou = inter / (a1 + a2 - inter + 1e-7)
        lbox = lbox + jnp.where(cnt > 0, jnp.sum((1.0 - iou) * mf) / cnt, 0.0)

        logits = ps[:, :, 5:]
        chan = lax.broadcasted_iota(jnp.int32, (_NA, _MP, _NC), 2)
        tmat = (chan == tcls[0][None, :, None]).astype(jnp.float32)
        per = (jnp.maximum(logits, 0.0) - logits * tmat
               + jnp.log1p(jnp.exp(-jnp.abs(logits))))
        lcls = lcls + jnp.where(
            cnt > 0, jnp.sum(per * mf[:, :, None]) / (cnt * _NC), 0.0)

        overwritten = jnp.zeros((_NA, _MP), jnp.bool_)
        for aj in range(_NA):
            cj = cell[aj:aj + 1, :]
            ej = mask[aj:aj + 1, :]
            oj = order[aj:aj + 1, :]
            eq = cell[:, :, None] == cj[0][None, None, :]
            later = oj[0][None, None, :] > order[:, :, None]
            hit = eq & later & ej[0][None, None, :]
            overwritten = overwritten | jnp.any(hit, axis=2)
        win = mask & ~overwritten
        corr = jnp.sum(jnp.where(win, ps[:, :, 4] * jnp.clip(iou, 0.0), 0.0))
        ncell = _B * _NA * ny * nx
        sref = (s0_ref, s1_ref, s2_ref)[lvl]
        lobj = lobj + (sref[0, 0] - corr) / ncell

    loss_ref[0, 0] = _BOX_GAIN * lbox + lobj + _CLS_GAIN * lcls
    lbox_ref[0, 0] = lbox
    lobj_ref[0, 0] = lobj
    lcls_ref[0, 0] = lcls


def _sparse_call(t_pad, g0, g1, g2, s0, s1, s2):
    smem = pl.BlockSpec(memory_space=pltpu.SMEM)
    return pl.pallas_call(
        _sparse_body,
        in_specs=[pl.BlockSpec()] * 4 + [smem] * 3,
        out_specs=[smem] * 4,
        out_shape=[jax.ShapeDtypeStruct((1, 1), jnp.float32)] * 4,
    )(t_pad, g0, g1, g2, s0, s1, s2)


def kernel(p0, p1, p2, targets):
    t_pad = jnp.zeros((8, _MP), jnp.float32).at[:6, :_M].set(targets.T)
    idx0, idx1, idx2, sub0, sub1, sub2 = _build_indices(t_pad)
    flat = lambda ix: ix[:4].reshape(_NPAD)
    tidx_all = jnp.concatenate([flat(idx0), flat(idx1), flat(idx2)])
    sub_all = jnp.concatenate([flat(sub0), flat(sub1), flat(sub2)])
    tbls = [p.reshape(-1, 8, 85) for p in (p0, p1, p2)]
    g0, g1, g2 = _gather_all(tbls, tidx_all, sub_all)
    flats = [p.reshape(-1, 85) for p in (p0, p1, p2)]
    s0 = _obj_softplus_sum(flats[0], 6400)
    s1 = _obj_softplus_sum(flats[1], 6400)
    s2 = _obj_softplus_sum(flats[2], 6400)
    r = lambda g: g.reshape(4, _MP, 8, 85)
    loss, lbox, lobj, lcls = _sparse_call(
        t_pad, r(g0), r(g1), r(g2), s0, s1, s2)
    return (loss[0, 0], lbox[0, 0], lobj[0, 0], lcls[0, 0])


# speedup vs baseline: 1.4449x; 1.0198x over previous
import functools

import jax
import jax.numpy as jnp
from jax import lax
from jax.experimental import pallas as pl
from jax.experimental.pallas import tpu as pltpu
from jax.experimental.pallas import tpu_sc as plsc

_NA = 3
_NC = 80
_ANCHOR_T = 4.0
_BOX_GAIN = 0.05
_CLS_GAIN = 0.5
_AG = (
    ((10.0 / 8.0, 13.0 / 8.0), (16.0 / 8.0, 30.0 / 8.0), (33.0 / 8.0, 23.0 / 8.0)),
    ((30.0 / 16.0, 61.0 / 16.0), (62.0 / 16.0, 45.0 / 16.0), (59.0 / 16.0, 119.0 / 16.0)),
    ((116.0 / 32.0, 90.0 / 32.0), (156.0 / 32.0, 198.0 / 32.0), (373.0 / 32.0, 326.0 / 32.0)),
)
_GRIDS = ((80, 80), (40, 40), (20, 20))
_B = 16
_M = 300
_MP = 320
_NPAD = 1280
_RPW = _NPAD // 32


def _meta_body(t_ref, i0_ref, i1_ref, i2_ref, s0_ref, s1_ref, s2_ref):
    tr = t_ref[...]
    col = lax.broadcasted_iota(jnp.int32, (8, _MP), 1)
    arow = lax.broadcasted_iota(jnp.int32, (8, _MP), 0)
    ok = (col < _M) & (arow < _NA)
    b = tr[0:1, :]
    x = tr[2:3, :]
    y = tr[3:4, :]
    bi = b.astype(jnp.int32)
    for (ny, nx), o_ref, s_ref in zip(
            _GRIDS, (i0_ref, i1_ref, i2_ref), (s0_ref, s1_ref, s2_ref)):
        gi = jnp.clip((x * nx).astype(jnp.int32), 0, nx - 1)
        gj = jnp.clip((y * ny).astype(jnp.int32), 0, ny - 1)
        idx = ((bi * _NA + arow) * ny + gj) * nx + gi
        o_ref[...] = jnp.where(ok, lax.shift_right_logical(idx, 3), 0)
        s_ref[...] = jnp.where(ok, jnp.bitwise_and(idx, 7), 0)


def _build_indices(t_pad):
    return pl.pallas_call(
        _meta_body,
        out_shape=[jax.ShapeDtypeStruct((8, _MP), jnp.int32)] * 6,
    )(t_pad)


_GQ = 64
_LSTEPS = _NPAD // _GQ


def _gather_all(tbls, tidx_all, sub_all):
    del sub_all

    def body(tref, *refs):
        o_ref = refs[_GQ]
        for q in range(_GQ):
            o_ref[q] = refs[q][0]

    def imap(q):
        return lambda i, tref: (tref[i * _GQ + q], 0, 0)

    def one(tbl, tidx):
        return pl.pallas_call(
            body,
            grid_spec=pltpu.PrefetchScalarGridSpec(
                num_scalar_prefetch=1,
                grid=(_NPAD // _GQ,),
                in_specs=[pl.BlockSpec((1, 8, 85), imap(q))
                          for q in range(_GQ)],
                out_specs=pl.BlockSpec((_GQ, 8, 85), lambda i, tref: (i, 0, 0)),
            ),
            out_shape=jax.ShapeDtypeStruct((_NPAD, 8, 85), jnp.float32),
        )(tidx, *([tbl] * _GQ))

    return [one(t, tidx_all[lvl * _NPAD:(lvl + 1) * _NPAD])
            for lvl, t in enumerate(tbls)]


def _obj_softplus_sum(tbl, blk):
    n = tbl.shape[0]

    def body(x_ref, o_ref):
        x = x_ref[:, 4:5]
        s = jnp.sum(jnp.maximum(x, 0.0) + jnp.log1p(jnp.exp(-jnp.abs(x))))

        @pl.when(pl.program_id(0) == 0)
        def _():
            o_ref[0, 0] = 0.0

        o_ref[0, 0] += s

    return pl.pallas_call(
        body,
        grid=(n // blk,),
        in_specs=[pl.BlockSpec((blk, 85), lambda i: (i, 0))],
        out_specs=pl.BlockSpec(memory_space=pltpu.SMEM),
        out_shape=jax.ShapeDtypeStruct((1, 1), jnp.float32),
    )(tbl)


def _sparse_body(t_ref, g0_ref, g1_ref, g2_ref, s0_ref, s1_ref, s2_ref,
                 loss_ref, lbox_ref, lobj_ref, lcls_ref):
    tr = t_ref[...]
    colv = lax.broadcasted_iota(jnp.int32, (1, _MP), 1)
    valid = colv < _M
    b = tr[0:1, :]
    cls = tr[1:2, :]
    x = tr[2:3, :]
    y = tr[3:4, :]
    w = tr[4:5, :]
    h = tr[5:6, :]
    bi = b.astype(jnp.int32)
    tcls = cls.astype(jnp.int32)
    arow3 = lax.broadcasted_iota(jnp.int32, (_NA, _MP), 0)
    order = arow3 * _M + lax.broadcasted_iota(jnp.int32, (_NA, _MP), 1)

    lbox = jnp.float32(0.0)
    lobj = jnp.float32(0.0)
    lcls = jnp.float32(0.0)
    for lvl, (ny, nx) in enumerate(_GRIDS):
        gx = x * nx
        gy = y * ny
        gw = w * nx
        gh = h * ny
        ar = lax.broadcasted_iota(jnp.int32, (_NA, 1), 0)
        ax0, ax1, ax2 = (_AG[lvl][a][0] for a in range(_NA))
        ay0, ay1, ay2 = (_AG[lvl][a][1] for a in range(_NA))
        awx = jnp.where(ar == 0, ax0, jnp.where(ar == 1, ax1, ax2))
        awy = jnp.where(ar == 0, ay0, jnp.where(ar == 1, ay1, ay2))
        rx = gw / awx
        ry = gh / awy
        mr = jnp.maximum(jnp.maximum(rx, 1.0 / (rx + 1e-9)),
                         jnp.maximum(ry, 1.0 / (ry + 1e-9)))
        mask = (mr < _ANCHOR_T) & valid
        mf = mask.astype(jnp.float32)
        cnt = jnp.sum(mf)
        gi = jnp.clip((x * nx).astype(jnp.int32), 0, nx - 1)
        gj = jnp.clip((y * ny).astype(jnp.int32), 0, ny - 1)
        gif = gi.astype(jnp.float32)
        gjf = gj.astype(jnp.float32)

        cell = ((bi * _NA + arow3) * ny + gj) * nx + gi
        sub = jnp.bitwise_and(cell, 7)
        g_ref = (g0_ref, g1_ref, g2_ref)[lvl]
        ps = jnp.zeros((_NA, _MP, 85), jnp.float32)
        for s in range(8):
            sel = (sub == s).astype(jnp.float32)
            ps = ps + g_ref[0:_NA, :, s, :] * sel[:, :, None]

        sig = jax.nn.sigmoid
        pxy_x = sig(ps[:, :, 0]) * 2.0 - 0.5 + gif
        pxy_y = sig(ps[:, :, 1]) * 2.0 - 0.5 + gjf
        pwh_x = (sig(ps[:, :, 2]) * 2.0) ** 2 * awx
        pwh_y = (sig(ps[:, :, 3]) * 2.0) ** 2 * awy

        b1x1 = pxy_x - pwh_x * 0.5
        b1y1 = pxy_y - pwh_y * 0.5
        b1x2 = pxy_x + pwh_x * 0.5
        b1y2 = pxy_y + pwh_y * 0.5
        b2x1 = gx - gw * 0.5
        b2y1 = gy - gh * 0.5
        b2x2 = gx + gw * 0.5
        b2y2 = gy + gh * 0.5
        iw = jnp.clip(jnp.minimum(b1x2, b2x2) - jnp.maximum(b1x1, b2x1), 0.0)
        ih = jnp.clip(jnp.minimum(b1y2, b2y2) - jnp.maximum(b1y1, b2y1), 0.0)
        inter = iw * ih
        a1 = jnp.clip(b1x2 - b1x1, 0.0) * jnp.clip(b1y2 - b1y1, 0.0)
        a2 = jnp.clip(b2x2 - b2x1, 0.0) * jnp.clip(b2y2 - b2y1, 0.0)
        iou = inter / (a1 + a2 - inter + 1e-7)
        lbox = lbox + jnp.where(cnt > 0, jnp.sum((1.0 - iou) * mf) / cnt, 0.0)

        logits = ps[:, :, 5:]
        chan = lax.broadcasted_iota(jnp.int32, (_NA, _MP, _NC), 2)
        tmat = (chan == tcls[0][None, :, None]).astype(jnp.float32)
        per = (jnp.maximum(logits, 0.0) - logits * tmat
               + jnp.log1p(jnp.exp(-jnp.abs(logits))))
        lcls = lcls + jnp.where(
            cnt > 0, jnp.sum(per * mf[:, :, None]) / (cnt * _NC), 0.0)

        overwritten = jnp.zeros((_NA, _MP), jnp.bool_)
        for aj in range(_NA):
            cj = cell[aj:aj + 1, :]
            ej = mask[aj:aj + 1, :]
            oj = order[aj:aj + 1, :]
            eq = cell[:, :, None] == cj[0][None, None, :]
            later = oj[0][None, None, :] > order[:, :, None]
            hit = eq & later & ej[0][None, None, :]
            overwritten = overwritten | jnp.any(hit, axis=2)
        win = mask & ~overwritten
        corr = jnp.sum(jnp.where(win, ps[:, :, 4] * jnp.clip(iou, 0.0), 0.0))
        ncell = _B * _NA * ny * nx
        sref = (s0_ref, s1_ref, s2_ref)[lvl]
        lobj = lobj + (sref[0, 0] - corr) / ncell

    loss_ref[0, 0] = _BOX_GAIN * lbox + lobj + _CLS_GAIN * lcls
    lbox_ref[0, 0] = lbox
    lobj_ref[0, 0] = lobj
    lcls_ref[0, 0] = lcls


def _sparse_call(t_pad, g0, g1, g2, s0, s1, s2):
    smem = pl.BlockSpec(memory_space=pltpu.SMEM)
    return pl.pallas_call(
        _sparse_body,
        in_specs=[pl.BlockSpec()] * 4 + [smem] * 3,
        out_specs=[smem] * 4,
        out_shape=[jax.ShapeDtypeStruct((1, 1), jnp.float32)] * 4,
    )(t_pad, g0, g1, g2, s0, s1, s2)


def kernel(p0, p1, p2, targets):
    t_pad = jnp.zeros((8, _MP), jnp.float32).at[:6, :_M].set(targets.T)
    idx0, idx1, idx2, sub0, sub1, sub2 = _build_indices(t_pad)
    flat = lambda ix: ix[:4].reshape(_NPAD)
    tidx_all = jnp.concatenate([flat(idx0), flat(idx1), flat(idx2)])
    sub_all = jnp.concatenate([flat(sub0), flat(sub1), flat(sub2)])
    tbls = [p.reshape(-1, 8, 85) for p in (p0, p1, p2)]
    g0, g1, g2 = _gather_all(tbls, tidx_all, sub_all)
    flats = [p.reshape(-1, 85) for p in (p0, p1, p2)]
    s0 = _obj_softplus_sum(flats[0], 6400)
    s1 = _obj_softplus_sum(flats[1], 6400)
    s2 = _obj_softplus_sum(flats[2], 6400)
    r = lambda g: g.reshape(4, _MP, 8, 85)
    loss, lbox, lobj, lcls = _sparse_call(
        t_pad, r(g0), r(g1), r(g2), s0, s1, s2)
    return (loss[0, 0], lbox[0, 0], lobj[0, 0], lcls[0, 0])


# speedup vs baseline: 1.4456x; 1.0005x over previous
import jax
import jax.numpy as jnp
from jax import lax
from jax.experimental import pallas as pl
from jax.experimental.pallas import tpu as pltpu

_NA = 3
_NC = 80
_ANCHOR_T = 4.0
_BOX_GAIN = 0.05
_CLS_GAIN = 0.5
_AG = (
    ((10.0 / 8.0, 13.0 / 8.0), (16.0 / 8.0, 30.0 / 8.0), (33.0 / 8.0, 23.0 / 8.0)),
    ((30.0 / 16.0, 61.0 / 16.0), (62.0 / 16.0, 45.0 / 16.0), (59.0 / 16.0, 119.0 / 16.0)),
    ((116.0 / 32.0, 90.0 / 32.0), (156.0 / 32.0, 198.0 / 32.0), (373.0 / 32.0, 326.0 / 32.0)),
)
_GRIDS = ((80, 80), (40, 40), (20, 20))
_B = 16
_M = 300
_MP = 320
_NPAD = 1280


def _meta_body(t_ref, i0_ref, i1_ref, i2_ref):
    tr = t_ref[...]
    col = lax.broadcasted_iota(jnp.int32, (8, _MP), 1)
    arow = lax.broadcasted_iota(jnp.int32, (8, _MP), 0)
    ok = (col < _M) & (arow < _NA)
    b = tr[0:1, :]
    x = tr[2:3, :]
    y = tr[3:4, :]
    bi = b.astype(jnp.int32)
    for (ny, nx), o_ref in zip(_GRIDS, (i0_ref, i1_ref, i2_ref)):
        gi = jnp.clip((x * nx).astype(jnp.int32), 0, nx - 1)
        gj = jnp.clip((y * ny).astype(jnp.int32), 0, ny - 1)
        idx = ((bi * _NA + arow) * ny + gj) * nx + gi
        o_ref[...] = jnp.where(ok, lax.shift_right_logical(idx, 3), 0)


def _build_indices(t_pad):
    return pl.pallas_call(
        _meta_body,
        out_shape=[jax.ShapeDtypeStruct((8, _MP), jnp.int32)] * 3,
    )(t_pad)


_GQ = 64
_LSTEPS = _NPAD // _GQ


def _gather_all(tbls, tidx_all):
    def body(tref, *refs):
        o_ref = refs[_GQ]
        for q in range(_GQ):
            o_ref[q] = refs[q][0]

    def imap(q):
        return lambda i, tref: (tref[i * _GQ + q], 0, 0)

    def one(tbl, tidx):
        return pl.pallas_call(
            body,
            grid_spec=pltpu.PrefetchScalarGridSpec(
                num_scalar_prefetch=1,
                grid=(_NPAD // _GQ,),
                in_specs=[pl.BlockSpec((1, 8, 85), imap(q))
                          for q in range(_GQ)],
                out_specs=pl.BlockSpec((_GQ, 8, 85), lambda i, tref: (i, 0, 0)),
            ),
            out_shape=jax.ShapeDtypeStruct((_NPAD, 8, 85), jnp.float32),
        )(tidx, *([tbl] * _GQ))

    return [one(t, tidx_all[lvl * _NPAD:(lvl + 1) * _NPAD])
            for lvl, t in enumerate(tbls)]


def _obj_softplus_sum(tbl, blk):
    n = tbl.shape[0]

    def body(x_ref, o_ref):
        x = x_ref[:, 4:5]
        s = jnp.sum(jnp.maximum(x, 0.0) + jnp.log1p(jnp.exp(-jnp.abs(x))))

        @pl.when(pl.program_id(0) == 0)
        def _():
            o_ref[0, 0] = 0.0

        o_ref[0, 0] += s

    return pl.pallas_call(
        body,
        grid=(n // blk,),
        in_specs=[pl.BlockSpec((blk, 85), lambda i: (i, 0))],
        out_specs=pl.BlockSpec(memory_space=pltpu.SMEM),
        out_shape=jax.ShapeDtypeStruct((1, 1), jnp.float32),
    )(tbl)


def _sparse_body(t_ref, g0_ref, g1_ref, g2_ref, s0_ref, s1_ref, s2_ref,
                 loss_ref, lbox_ref, lobj_ref, lcls_ref):
    tr = t_ref[...]
    colv = lax.broadcasted_iota(jnp.int32, (1, _MP), 1)
    valid = colv < _M
    b = tr[0:1, :]
    cls = tr[1:2, :]
    x = tr[2:3, :]
    y = tr[3:4, :]
    w = tr[4:5, :]
    h = tr[5:6, :]
    bi = b.astype(jnp.int32)
    tcls = cls.astype(jnp.int32)
    arow3 = lax.broadcasted_iota(jnp.int32, (_NA, _MP), 0)
    order = arow3 * _M + lax.broadcasted_iota(jnp.int32, (_NA, _MP), 1)

    lbox = jnp.float32(0.0)
    lobj = jnp.float32(0.0)
    lcls = jnp.float32(0.0)
    for lvl, (ny, nx) in enumerate(_GRIDS):
        gx = x * nx
        gy = y * ny
        gw = w * nx
        gh = h * ny
        ar = lax.broadcasted_iota(jnp.int32, (_NA, 1), 0)
        ax0, ax1, ax2 = (_AG[lvl][a][0] for a in range(_NA))
        ay0, ay1, ay2 = (_AG[lvl][a][1] for a in range(_NA))
        awx = jnp.where(ar == 0, ax0, jnp.where(ar == 1, ax1, ax2))
        awy = jnp.where(ar == 0, ay0, jnp.where(ar == 1, ay1, ay2))
        rx = gw / awx
        ry = gh / awy
        mr = jnp.maximum(jnp.maximum(rx, 1.0 / (rx + 1e-9)),
                         jnp.maximum(ry, 1.0 / (ry + 1e-9)))
        mask = (mr < _ANCHOR_T) & valid
        mf = mask.astype(jnp.float32)
        cnt = jnp.sum(mf)
        gi = jnp.clip((x * nx).astype(jnp.int32), 0, nx - 1)
        gj = jnp.clip((y * ny).astype(jnp.int32), 0, ny - 1)
        gif = gi.astype(jnp.float32)
        gjf = gj.astype(jnp.float32)

        cell = ((bi * _NA + arow3) * ny + gj) * nx + gi
        sub = jnp.bitwise_and(cell, 7)
        g_ref = (g0_ref, g1_ref, g2_ref)[lvl]
        ps = jnp.zeros((_NA, _MP, 85), jnp.float32)
        for s in range(8):
            sel = (sub == s).astype(jnp.float32)
            ps = ps + g_ref[0:_NA, :, s, :] * sel[:, :, None]

        sig = jax.nn.sigmoid
        pxy_x = sig(ps[:, :, 0]) * 2.0 - 0.5 + gif
        pxy_y = sig(ps[:, :, 1]) * 2.0 - 0.5 + gjf
        pwh_x = (sig(ps[:, :, 2]) * 2.0) ** 2 * awx
        pwh_y = (sig(ps[:, :, 3]) * 2.0) ** 2 * awy

        b1x1 = pxy_x - pwh_x * 0.5
        b1y1 = pxy_y - pwh_y * 0.5
        b1x2 = pxy_x + pwh_x * 0.5
        b1y2 = pxy_y + pwh_y * 0.5
        b2x1 = gx - gw * 0.5
        b2y1 = gy - gh * 0.5
        b2x2 = gx + gw * 0.5
        b2y2 = gy + gh * 0.5
        iw = jnp.clip(jnp.minimum(b1x2, b2x2) - jnp.maximum(b1x1, b2x1), 0.0)
        ih = jnp.clip(jnp.minimum(b1y2, b2y2) - jnp.maximum(b1y1, b2y1), 0.0)
        inter = iw * ih
        a1 = jnp.clip(b1x2 - b1x1, 0.0) * jnp.clip(b1y2 - b1y1, 0.0)
        a2 = jnp.clip(b2x2 - b2x1, 0.0) * jnp.clip(b2y2 - b2y1, 0.0)
        iou = inter / (a1 + a2 - inter + 1e-7)
        lbox = lbox + jnp.where(cnt > 0, jnp.sum((1.0 - iou) * mf) / cnt, 0.0)

        logits = ps[:, :, 5:]
        chan = lax.broadcasted_iota(jnp.int32, (_NA, _MP, _NC), 2)
        tmat = (chan == tcls[0][None, :, None]).astype(jnp.float32)
        per = (jnp.maximum(logits, 0.0) - logits * tmat
               + jnp.log1p(jnp.exp(-jnp.abs(logits))))
        lcls = lcls + jnp.where(
            cnt > 0, jnp.sum(per * mf[:, :, None]) / (cnt * _NC), 0.0)

        overwritten = jnp.zeros((_NA, _MP), jnp.bool_)
        for aj in range(_NA):
            cj = cell[aj:aj + 1, :]
            ej = mask[aj:aj + 1, :]
            oj = order[aj:aj + 1, :]
            eq = cell[:, :, None] == cj[0][None, None, :]
            later = oj[0][None, None, :] > order[:, :, None]
            hit = eq & later & ej[0][None, None, :]
            overwritten = overwritten | jnp.any(hit, axis=2)
        win = mask & ~overwritten
        corr = jnp.sum(jnp.where(win, ps[:, :, 4] * jnp.clip(iou, 0.0), 0.0))
        ncell = _B * _NA * ny * nx
        sref = (s0_ref, s1_ref, s2_ref)[lvl]
        lobj = lobj + (sref[0, 0] - corr) / ncell

    loss_ref[0, 0] = _BOX_GAIN * lbox + lobj + _CLS_GAIN * lcls
    lbox_ref[0, 0] = lbox
    lobj_ref[0, 0] = lobj
    lcls_ref[0, 0] = lcls


def _sparse_call(t_pad, g0, g1, g2, s0, s1, s2):
    smem = pl.BlockSpec(memory_space=pltpu.SMEM)
    return pl.pallas_call(
        _sparse_body,
        in_specs=[pl.BlockSpec()] * 4 + [smem] * 3,
        out_specs=[smem] * 4,
        out_shape=[jax.ShapeDtypeStruct((1, 1), jnp.float32)] * 4,
    )(t_pad, g0, g1, g2, s0, s1, s2)


def kernel(p0, p1, p2, targets):
    t_pad = jnp.zeros((8, _MP), jnp.float32).at[:6, :_M].set(targets.T)
    idx0, idx1, idx2 = _build_indices(t_pad)
    flat = lambda ix: ix[:4].reshape(_NPAD)
    tidx_all = jnp.concatenate([flat(idx0), flat(idx1), flat(idx2)])
    tbls = [p.reshape(-1, 8, 85) for p in (p0, p1, p2)]
    g0, g1, g2 = _gather_all(tbls, tidx_all)
    flats = [p.reshape(-1, 85) for p in (p0, p1, p2)]
    s0 = _obj_softplus_sum(flats[0], 6400)
    s1 = _obj_softplus_sum(flats[1], 6400)
    s2 = _obj_softplus_sum(flats[2], 6400)
    r = lambda g: g.reshape(4, _MP, 8, 85)
    loss, lbox, lobj, lcls = _sparse_call(
        t_pad, r(g0), r(g1), r(g2), s0, s1, s2)
    return (loss[0, 0], lbox[0, 0], lobj[0, 0], lcls[0, 0])
